# baseline (speedup 1.0000x reference)
import functools

import jax
import jax.numpy as jnp
import numpy as np
from jax import lax
from jax.experimental import pallas as pl
from jax.experimental.pallas import tpu as pltpu
from jax.experimental.pallas import tpu_sc as plsc

_NPOINTS = 512
_RADIUS = 0.2
_NSAMPLE = 32
_B = 4
_NVOX = 4096
_NALL = 16384
_R2 = np.float32(_RADIUS * _RADIUS)

_ROWS = 8
_COLS = _NVOX // _ROWS
_QROWS = 8
_QCOLS = _NPOINTS // _QROWS


def _fps_body(xvp_ref, xs_smem, nxyz_ref, idx_ref):
    X = xvp_ref[0]
    Y = xvp_ref[1]
    Z = xvp_ref[2]
    jdx = ((lax.broadcasted_iota(jnp.int32, (8, 2048), 0) % 2) * 2048
           + lax.broadcasted_iota(jnp.int32, (8, 2048), 1))
    pos = (lax.broadcasted_iota(jnp.int32, (_QROWS, _QCOLS), 0) * _QCOLS
           + lax.broadcasted_iota(jnp.int32, (_QROWS, _QCOLS), 1))
    evenv = (lax.broadcasted_iota(jnp.int32, (8, 128), 0) % 2) == 0
    pcol = lax.broadcasted_iota(jnp.int32, (8, 1), 0) // 2
    zf = jnp.float32(0.0)

    def red_pair(a, jop):
        h = jop(a[:, :1024], a[:, 1024:])
        h = jop(h[:, :512], h[:, 512:])
        h = jop(h[:, :256], h[:, 256:])
        h = jop(h[:, :128], h[:, 128:])
        for sh in (1, 2, 4, 8, 16, 32, 64):
            h = jop(h, pltpu.roll(h, sh, axis=1))
        up = pltpu.roll(h, 7, axis=0)
        dn = pltpu.roll(h, 1, axis=0)
        return jop(h, jnp.where(evenv, up, dn))

    def col4(vals):
        return jnp.where(pcol == 0, vals[0],
                         jnp.where(pcol == 1, vals[1],
                                   jnp.where(pcol == 2, vals[2], vals[3])))

    def centroid(b, last_b):
        return tuple(xs_smem[(b * 3 + c) * _NVOX + last_b] for c in range(3))

    dists0 = jnp.full((8, 2048), 1e10, jnp.float32)
    lasts0 = tuple(jnp.int32(0) for _ in range(_B))
    nx0 = tuple(tuple(jnp.zeros((_QROWS, _QCOLS), jnp.float32) for _ in range(3))
                for _ in range(_B))
    ix0 = tuple(jnp.zeros((_QROWS, _QCOLS), jnp.int32) for _ in range(_B))

    def body(i, carry):
        dists, lasts, nxa, ixa = carry
        oh_prev = pos == (i - 1)
        oh_cur = pos == i
        cs = [centroid(b, lasts[b]) for b in range(_B)]
        cxc = col4([cs[b][0] for b in range(_B)])
        cyc = col4([cs[b][1] for b in range(_B)])
        czc = col4([cs[b][2] for b in range(_B)])
        dx = X - cxc
        dy = Y - cyc
        dz = Z - czc
        d = dx * dx + dy * dy + dz * dz
        db = jnp.minimum(dists, d)
        mxp = red_pair(db, jnp.maximum)
        mxb = jnp.concatenate([mxp] * 16, axis=1)
        farp = red_pair(jnp.where(db == mxb, jdx, _NVOX), jnp.minimum)
        fars = tuple(farp[2 * b, 0] for b in range(_B))
        nnx = tuple(tuple(nxa[b][c] + jnp.where(oh_prev, cs[b][c], zf)
                          for c in range(3)) for b in range(_B))
        nix = tuple(ixa[b] + jnp.where(oh_cur, fars[b], jnp.int32(0))
                    for b in range(_B))
        return db, fars, nnx, nix

    dists, lasts, nxa, ixa = lax.fori_loop(
        1, _NPOINTS, body, (dists0, lasts0, nx0, ix0))

    oh_last = pos == (_NPOINTS - 1)
    for b in range(_B):
        cx, cy, cz = centroid(b, lasts[b])
        for c, v in enumerate((cx, cy, cz)):
            nxyz_ref[b, c] = nxa[b][c] + jnp.where(oh_last, v, zf)
        idx_ref[b] = ixa[b]


_fps_call = pl.pallas_call(
    _fps_body,
    in_specs=[
        pl.BlockSpec(memory_space=pltpu.VMEM),
        pl.BlockSpec(memory_space=pltpu.SMEM),
    ],
    out_shape=(
        jax.ShapeDtypeStruct((_B, 3, _QROWS, _QCOLS), jnp.float32),
        jax.ShapeDtypeStruct((_B, _QROWS, _QCOLS), jnp.int32),
    ),
)


@functools.cache
def _make_sc_kernel():
    return pl.kernel(
        _sc_body,
        out_type=(
            jax.ShapeDtypeStruct((_B * _NPOINTS * 3,), jnp.float32),
            jax.ShapeDtypeStruct((_B * _NPOINTS * 3,), jnp.float32),
            jax.ShapeDtypeStruct((_B * 3 * _NPOINTS * _NSAMPLE,), jnp.float32),
            jax.ShapeDtypeStruct((_B * 6 * _NPOINTS * _NSAMPLE,), jnp.float32),
        ),
        mesh=plsc.VectorSubcoreMesh(core_axis_name="c", subcore_axis_name="s",
                                    num_cores=2, num_subcores=16),
        compiler_params=pltpu.CompilerParams(needs_layout_passes=False),
        scratch_types=[
            pltpu.VMEM((_NALL,), jnp.float32),
            pltpu.VMEM((_NALL,), jnp.float32),
            pltpu.VMEM((_NALL,), jnp.float32),
            pltpu.VMEM((_NALL,), jnp.float32),
            pltpu.VMEM((_NALL,), jnp.float32),
            pltpu.VMEM((_NALL,), jnp.float32),
            pltpu.VMEM((_NVOX,), jnp.float32),
            pltpu.VMEM((_NVOX,), jnp.float32),
            pltpu.VMEM((_NVOX,), jnp.float32),
            pltpu.VMEM((64,), jnp.int32),
            pltpu.VMEM((192,), jnp.float32),
            pltpu.VMEM((64,), jnp.int32),
            pltpu.VMEM((3 * 16 * _NSAMPLE,), jnp.float32),
            pltpu.VMEM((3 * 16 * _NSAMPLE,), jnp.float32),
            pltpu.VMEM((48,), jnp.float32),
            pltpu.VMEM((48,), jnp.float32),
            pltpu.SMEM((1,), jnp.int32),
        ],
    )


def _sc_body(xa, nrm, nvox, nxc, fpsi,
             nxyz_o, nnorm_o, feat_o, nfn_o,
             Xr, Yr, Zr, NXr, NYr, NZr, VNX, VNY, VNZ,
             fidx, nxq, idxbuf, fbuf, nbuf, nxbuf, nnbuf, cnt_ref):
    wid = lax.axis_index("s") * 2 + lax.axis_index("c")
    b = wid // 8
    r = wid % 8

    for c, ref in enumerate((Xr, Yr, Zr)):
        off = pl.multiple_of((b * 3 + c) * _NALL, _NALL)
        pltpu.sync_copy(xa.at[pl.ds(off, _NALL)], ref)
    for c, ref in enumerate((NXr, NYr, NZr)):
        off = pl.multiple_of((b * 3 + c) * _NALL, _NALL)
        pltpu.sync_copy(nrm.at[pl.ds(off, _NALL)], ref)
    for c, ref in enumerate((VNX, VNY, VNZ)):
        off = pl.multiple_of((b * 3 + c) * _NVOX, _NVOX)
        pltpu.sync_copy(nvox.at[pl.ds(off, _NVOX)], ref)
    pltpu.sync_copy(fpsi.at[pl.ds(pl.multiple_of(wid * 64, 64), 64)], fidx)
    for c in range(3):
        off = pl.multiple_of((b * 3 + c) * _NPOINTS + r * 64, 64)
        pltpu.sync_copy(nxc.at[pl.ds(off, 64)], nxq.at[pl.ds(c * 64, 64)])

    i16 = lax.iota(jnp.int32, 16)
    zeros16 = jnp.zeros((16,), jnp.int32)

    def blk_body(blk, _):
        q0l = blk * 16
        qg = r * 64 + q0l
        qxv = nxq[pl.ds(q0l, 16)]
        qyv = nxq[pl.ds(64 + q0l, 16)]
        qzv = nxq[pl.ds(128 + q0l, 16)]
        for qi in range(16):
            qx = qxv[qi]
            qy = qyv[qi]
            qz = qzv[qi]
            idxbuf[pl.ds(0, 16)] = zeros16
            idxbuf[pl.ds(16, 16)] = zeros16
            idxbuf[pl.ds(32, 16)] = zeros16

            cnt_ref[0] = 0

            def seg_body(sg, _):
                @pl.when(cnt_ref[0] < _NSAMPLE)
                def _():
                    def chunk_body(ck, cnt):
                        cbase = sg * 256 + ck * 16
                        xsv = Xr[pl.ds(cbase, 16)]
                        ysv = Yr[pl.ds(cbase, 16)]
                        zsv = Zr[pl.ds(cbase, 16)]
                        dx = xsv - qx
                        dy = ysv - qy
                        dz = zsv - qz
                        d2 = dx * dx + dy * dy + dz * dz
                        m = d2 < _R2
                        jv = i16 + cbase
                        csum = plsc.cumsum(m.astype(jnp.int32))
                        pos = jnp.minimum(cnt + csum - 1, 63)
                        pos = jnp.where(m, pos, 63)
                        plsc.store_scatter(idxbuf, [pos], jv)
                        return cnt + csum[15]

                    cnt_ref[0] = lax.fori_loop(0, 16, chunk_body, cnt_ref[0])
                return 0

            lax.fori_loop(0, _NALL // 256, seg_body, 0)
            cnt = cnt_ref[0]
            v0 = idxbuf[pl.ds(0, 16)][0]
            for k in (0, 16):
                lane = i16 + k
                vec = idxbuf[pl.ds(k, 16)]
                sel = jnp.where(lane >= cnt, v0, vec)
                gx = plsc.load_gather(Xr, [sel]) - qx
                gy = plsc.load_gather(Yr, [sel]) - qy
                gz = plsc.load_gather(Zr, [sel]) - qz
                fbuf[pl.ds(qi * _NSAMPLE + k, 16)] = gx
                fbuf[pl.ds(512 + qi * _NSAMPLE + k, 16)] = gy
                fbuf[pl.ds(1024 + qi * _NSAMPLE + k, 16)] = gz
                nbuf[pl.ds(qi * _NSAMPLE + k, 16)] = plsc.load_gather(NXr, [sel])
                nbuf[pl.ds(512 + qi * _NSAMPLE + k, 16)] = plsc.load_gather(NYr, [sel])
                nbuf[pl.ds(1024 + qi * _NSAMPLE + k, 16)] = plsc.load_gather(NZr, [sel])

        fi = fidx[pl.ds(q0l, 16)]
        lanes3 = i16 * 3
        for c, (tab, qv) in enumerate(((VNX, qxv), (VNY, qyv), (VNZ, qzv))):
            g = plsc.load_gather(tab, [fi])
            plsc.store_scatter(nnbuf, [lanes3 + c], g)
            plsc.store_scatter(nxbuf, [lanes3 + c], qv)
        row_off = pl.multiple_of((b * _NPOINTS + qg) * 3, 48)
        pltpu.sync_copy(nnbuf, nnorm_o.at[pl.ds(row_off, 48)])
        pltpu.sync_copy(nxbuf, nxyz_o.at[pl.ds(row_off, 48)])
        for c in range(3):
            fsrc = fbuf.at[pl.ds(c * 512, 512)]
            nsrc = nbuf.at[pl.ds(c * 512, 512)]
            foff = pl.multiple_of(((b * 3 + c) * _NPOINTS + qg) * _NSAMPLE, 512)
            pltpu.sync_copy(fsrc, feat_o.at[pl.ds(foff, 512)])
            noff = pl.multiple_of(((b * 6 + c) * _NPOINTS + qg) * _NSAMPLE, 512)
            pltpu.sync_copy(fsrc, nfn_o.at[pl.ds(noff, 512)])
            noff2 = pl.multiple_of(((b * 6 + c + 3) * _NPOINTS + qg) * _NSAMPLE, 512)
            pltpu.sync_copy(nsrc, nfn_o.at[pl.ds(noff2, 512)])
        return 0

    lax.fori_loop(0, 4, blk_body, 0)


def kernel(xyz_all, normals, xyz_voxel, normals_voxel):
    xvp = xyz_voxel.transpose(2, 0, 1).reshape(3, 8, 2048)
    xv_sm = xyz_voxel.transpose(0, 2, 1).reshape(-1)
    nx_cm, fps_i = _fps_call(xvp, xv_sm)
    nxc = nx_cm.reshape(-1)
    xa_cm = xyz_all.transpose(0, 2, 1).reshape(-1)
    nrm_f = normals.reshape(-1)
    nvox_cm = normals_voxel.transpose(0, 2, 1).reshape(-1)
    new_xyz, new_normals, feat, nfn = _make_sc_kernel()(
        xa_cm, nrm_f, nvox_cm, nxc, fps_i.reshape(-1))
    return (new_xyz.reshape(_B, _NPOINTS, 3),
            new_normals.reshape(_B, _NPOINTS, 3),
            feat.reshape(_B, 3, _NPOINTS, _NSAMPLE),
            nfn.reshape(_B, 6, _NPOINTS, _NSAMPLE))


# speedup vs baseline: 1.6971x; 1.6971x over previous
import functools

import jax
import jax.numpy as jnp
import numpy as np
from jax import lax
from jax.experimental import pallas as pl
from jax.experimental.pallas import tpu as pltpu
from jax.experimental.pallas import tpu_sc as plsc

_NPOINTS = 512
_RADIUS = 0.2
_NSAMPLE = 32
_B = 4
_NVOX = 4096
_NALL = 16384
_R2 = np.float32(_RADIUS * _RADIUS)

_ROWS = 8
_COLS = _NVOX // _ROWS
_QROWS = 8
_QCOLS = _NPOINTS // _QROWS


def _fps_body(xvp_ref, xs_smem, nxyz_ref, idx_ref):
    X = xvp_ref[0]
    Y = xvp_ref[1]
    Z = xvp_ref[2]
    jdx = ((lax.broadcasted_iota(jnp.int32, (8, 2048), 0) % 2) * 2048
           + lax.broadcasted_iota(jnp.int32, (8, 2048), 1))
    pos = (lax.broadcasted_iota(jnp.int32, (_QROWS, _QCOLS), 0) * _QCOLS
           + lax.broadcasted_iota(jnp.int32, (_QROWS, _QCOLS), 1))
    evenv = (lax.broadcasted_iota(jnp.int32, (8, 128), 0) % 2) == 0
    pcol = lax.broadcasted_iota(jnp.int32, (8, 1), 0) // 2
    zf = jnp.float32(0.0)

    def col4(vals):
        return jnp.where(pcol == 0, vals[0],
                         jnp.where(pcol == 1, vals[1],
                                   jnp.where(pcol == 2, vals[2], vals[3])))

    def centroid(b, last_b):
        return tuple(xs_smem[(b * 3 + c) * _NVOX + last_b] for c in range(3))

    dists0 = jnp.full((8, 2048), 1e10, jnp.float32)
    lasts0 = tuple(jnp.int32(0) for _ in range(_B))
    nx0 = tuple(tuple(jnp.zeros((_QROWS, _QCOLS), jnp.float32) for _ in range(3))
                for _ in range(_B))
    ix0 = tuple(jnp.zeros((_QROWS, _QCOLS), jnp.int32) for _ in range(_B))

    def body(i, carry):
        dists, lasts, nxa, ixa = carry
        oh_prev = pos == (i - 1)
        oh_cur = pos == i
        cs = [centroid(b, lasts[b]) for b in range(_B)]
        cxc = col4([cs[b][0] for b in range(_B)])
        cyc = col4([cs[b][1] for b in range(_B)])
        czc = col4([cs[b][2] for b in range(_B)])
        dx = X - cxc
        dy = Y - cyc
        dz = Z - czc
        d = dx * dx + dy * dy + dz * dz
        db = jnp.minimum(dists, d)
        db3 = db.reshape(_B, 2, 2048)
        mx4 = jnp.max(db3, axis=(1, 2), keepdims=True)
        mxb = jnp.broadcast_to(mx4, (_B, 2, 2048)).reshape(8, 2048)
        sel = jnp.where(db == mxb, jdx, _NVOX)
        far4 = jnp.min(sel.reshape(_B, 2, 2048), axis=(1, 2))
        fars = tuple(far4[b] for b in range(_B))
        nnx = tuple(tuple(nxa[b][c] + jnp.where(oh_prev, cs[b][c], zf)
                          for c in range(3)) for b in range(_B))
        nix = tuple(ixa[b] + jnp.where(oh_cur, fars[b], jnp.int32(0))
                    for b in range(_B))
        return db, fars, nnx, nix

    dists, lasts, nxa, ixa = lax.fori_loop(
        1, _NPOINTS, body, (dists0, lasts0, nx0, ix0))

    oh_last = pos == (_NPOINTS - 1)
    for b in range(_B):
        cx, cy, cz = centroid(b, lasts[b])
        for c, v in enumerate((cx, cy, cz)):
            nxyz_ref[b, c] = nxa[b][c] + jnp.where(oh_last, v, zf)
        idx_ref[b] = ixa[b]


_fps_call = pl.pallas_call(
    _fps_body,
    in_specs=[
        pl.BlockSpec(memory_space=pltpu.VMEM),
        pl.BlockSpec(memory_space=pltpu.SMEM),
    ],
    out_shape=(
        jax.ShapeDtypeStruct((_B, 3, _QROWS, _QCOLS), jnp.float32),
        jax.ShapeDtypeStruct((_B, _QROWS, _QCOLS), jnp.int32),
    ),
)


@functools.cache
def _make_sc_kernel():
    return pl.kernel(
        _sc_body,
        out_type=(
            jax.ShapeDtypeStruct((_B * _NPOINTS * 3,), jnp.float32),
            jax.ShapeDtypeStruct((_B * _NPOINTS * 3,), jnp.float32),
            jax.ShapeDtypeStruct((_B * 3 * _NPOINTS * _NSAMPLE,), jnp.float32),
            jax.ShapeDtypeStruct((_B * 6 * _NPOINTS * _NSAMPLE,), jnp.float32),
        ),
        mesh=plsc.VectorSubcoreMesh(core_axis_name="c", subcore_axis_name="s",
                                    num_cores=2, num_subcores=16),
        compiler_params=pltpu.CompilerParams(needs_layout_passes=False),
        scratch_types=[
            pltpu.VMEM((_NALL,), jnp.float32),
            pltpu.VMEM((_NALL,), jnp.float32),
            pltpu.VMEM((_NALL,), jnp.float32),
            pltpu.VMEM((_NALL,), jnp.float32),
            pltpu.VMEM((_NALL,), jnp.float32),
            pltpu.VMEM((_NALL,), jnp.float32),
            pltpu.VMEM((_NVOX,), jnp.float32),
            pltpu.VMEM((_NVOX,), jnp.float32),
            pltpu.VMEM((_NVOX,), jnp.float32),
            pltpu.VMEM((64,), jnp.int32),
            pltpu.VMEM((192,), jnp.float32),
            pltpu.VMEM((64,), jnp.int32),
            pltpu.VMEM((3 * 16 * _NSAMPLE,), jnp.float32),
            pltpu.VMEM((3 * 16 * _NSAMPLE,), jnp.float32),
            pltpu.VMEM((48,), jnp.float32),
            pltpu.VMEM((48,), jnp.float32),
            pltpu.SMEM((1,), jnp.int32),
        ],
    )


def _sc_body(xa, nrm, nvox, nxc, fpsi,
             nxyz_o, nnorm_o, feat_o, nfn_o,
             Xr, Yr, Zr, NXr, NYr, NZr, VNX, VNY, VNZ,
             fidx, nxq, idxbuf, fbuf, nbuf, nxbuf, nnbuf, cnt_ref):
    wid = lax.axis_index("s") * 2 + lax.axis_index("c")
    b = wid // 8
    r = wid % 8

    for c, ref in enumerate((Xr, Yr, Zr)):
        off = pl.multiple_of((b * 3 + c) * _NALL, _NALL)
        pltpu.sync_copy(xa.at[pl.ds(off, _NALL)], ref)
    for c, ref in enumerate((NXr, NYr, NZr)):
        off = pl.multiple_of((b * 3 + c) * _NALL, _NALL)
        pltpu.sync_copy(nrm.at[pl.ds(off, _NALL)], ref)
    for c, ref in enumerate((VNX, VNY, VNZ)):
        off = pl.multiple_of((b * 3 + c) * _NVOX, _NVOX)
        pltpu.sync_copy(nvox.at[pl.ds(off, _NVOX)], ref)
    pltpu.sync_copy(fpsi.at[pl.ds(pl.multiple_of(wid * 64, 64), 64)], fidx)
    for c in range(3):
        off = pl.multiple_of((b * 3 + c) * _NPOINTS + r * 64, 64)
        pltpu.sync_copy(nxc.at[pl.ds(off, 64)], nxq.at[pl.ds(c * 64, 64)])

    i16 = lax.iota(jnp.int32, 16)
    zeros16 = jnp.zeros((16,), jnp.int32)

    def blk_body(blk, _):
        q0l = blk * 16
        qg = r * 64 + q0l
        qxv = nxq[pl.ds(q0l, 16)]
        qyv = nxq[pl.ds(64 + q0l, 16)]
        qzv = nxq[pl.ds(128 + q0l, 16)]
        for qi in range(16):
            qx = qxv[qi]
            qy = qyv[qi]
            qz = qzv[qi]
            idxbuf[pl.ds(0, 16)] = zeros16
            idxbuf[pl.ds(16, 16)] = zeros16
            idxbuf[pl.ds(32, 16)] = zeros16

            cnt_ref[0] = 0

            def seg_body(sg, _):
                @pl.when(cnt_ref[0] < _NSAMPLE)
                def _():
                    def chunk_body(ck, cnt):
                        cbase = sg * 256 + ck * 16
                        xsv = Xr[pl.ds(cbase, 16)]
                        ysv = Yr[pl.ds(cbase, 16)]
                        zsv = Zr[pl.ds(cbase, 16)]
                        dx = xsv - qx
                        dy = ysv - qy
                        dz = zsv - qz
                        d2 = dx * dx + dy * dy + dz * dz
                        m = d2 < _R2
                        jv = i16 + cbase
                        csum = plsc.cumsum(m.astype(jnp.int32))
                        pos = jnp.minimum(cnt + csum - 1, 63)
                        pos = jnp.where(m, pos, 63)
                        plsc.store_scatter(idxbuf, [pos], jv)
                        return cnt + csum[15]

                    cnt_ref[0] = lax.fori_loop(0, 16, chunk_body, cnt_ref[0])
                return 0

            lax.fori_loop(0, _NALL // 256, seg_body, 0)
            cnt = cnt_ref[0]
            v0 = idxbuf[pl.ds(0, 16)][0]
            for k in (0, 16):
                lane = i16 + k
                vec = idxbuf[pl.ds(k, 16)]
                sel = jnp.where(lane >= cnt, v0, vec)
                gx = plsc.load_gather(Xr, [sel]) - qx
                gy = plsc.load_gather(Yr, [sel]) - qy
                gz = plsc.load_gather(Zr, [sel]) - qz
                fbuf[pl.ds(qi * _NSAMPLE + k, 16)] = gx
                fbuf[pl.ds(512 + qi * _NSAMPLE + k, 16)] = gy
                fbuf[pl.ds(1024 + qi * _NSAMPLE + k, 16)] = gz
                nbuf[pl.ds(qi * _NSAMPLE + k, 16)] = plsc.load_gather(NXr, [sel])
                nbuf[pl.ds(512 + qi * _NSAMPLE + k, 16)] = plsc.load_gather(NYr, [sel])
                nbuf[pl.ds(1024 + qi * _NSAMPLE + k, 16)] = plsc.load_gather(NZr, [sel])

        fi = fidx[pl.ds(q0l, 16)]
        lanes3 = i16 * 3
        for c, (tab, qv) in enumerate(((VNX, qxv), (VNY, qyv), (VNZ, qzv))):
            g = plsc.load_gather(tab, [fi])
            plsc.store_scatter(nnbuf, [lanes3 + c], g)
            plsc.store_scatter(nxbuf, [lanes3 + c], qv)
        row_off = pl.multiple_of((b * _NPOINTS + qg) * 3, 48)
        pltpu.sync_copy(nnbuf, nnorm_o.at[pl.ds(row_off, 48)])
        pltpu.sync_copy(nxbuf, nxyz_o.at[pl.ds(row_off, 48)])
        for c in range(3):
            fsrc = fbuf.at[pl.ds(c * 512, 512)]
            nsrc = nbuf.at[pl.ds(c * 512, 512)]
            foff = pl.multiple_of(((b * 3 + c) * _NPOINTS + qg) * _NSAMPLE, 512)
            pltpu.sync_copy(fsrc, feat_o.at[pl.ds(foff, 512)])
            noff = pl.multiple_of(((b * 6 + c) * _NPOINTS + qg) * _NSAMPLE, 512)
            pltpu.sync_copy(fsrc, nfn_o.at[pl.ds(noff, 512)])
            noff2 = pl.multiple_of(((b * 6 + c + 3) * _NPOINTS + qg) * _NSAMPLE, 512)
            pltpu.sync_copy(nsrc, nfn_o.at[pl.ds(noff2, 512)])
        return 0

    lax.fori_loop(0, 4, blk_body, 0)


def kernel(xyz_all, normals, xyz_voxel, normals_voxel):
    xvp = xyz_voxel.transpose(2, 0, 1).reshape(3, 8, 2048)
    xv_sm = xyz_voxel.transpose(0, 2, 1).reshape(-1)
    nx_cm, fps_i = _fps_call(xvp, xv_sm)
    nxc = nx_cm.reshape(-1)
    xa_cm = xyz_all.transpose(0, 2, 1).reshape(-1)
    nrm_f = normals.reshape(-1)
    nvox_cm = normals_voxel.transpose(0, 2, 1).reshape(-1)
    new_xyz, new_normals, feat, nfn = _make_sc_kernel()(
        xa_cm, nrm_f, nvox_cm, nxc, fps_i.reshape(-1))
    return (new_xyz.reshape(_B, _NPOINTS, 3),
            new_normals.reshape(_B, _NPOINTS, 3),
            feat.reshape(_B, 3, _NPOINTS, _NSAMPLE),
            nfn.reshape(_B, 6, _NPOINTS, _NSAMPLE))


# speedup vs baseline: 1.7237x; 1.0157x over previous
import functools

import jax
import jax.numpy as jnp
import numpy as np
from jax import lax
from jax.experimental import pallas as pl
from jax.experimental.pallas import tpu as pltpu
from jax.experimental.pallas import tpu_sc as plsc

_NPOINTS = 512
_RADIUS = 0.2
_NSAMPLE = 32
_B = 4
_NVOX = 4096
_NALL = 16384
_R2 = np.float32(_RADIUS * _RADIUS)

_ROWS = 8
_COLS = _NVOX // _ROWS
_QROWS = 8
_QCOLS = _NPOINTS // _QROWS


def _fps_body(xvp_ref, xs_smem, nxyz_ref, idx_ref):
    X = xvp_ref[0]
    Y = xvp_ref[1]
    Z = xvp_ref[2]
    jdx = ((lax.broadcasted_iota(jnp.int32, (8, 2048), 0) % 2) * 2048
           + lax.broadcasted_iota(jnp.int32, (8, 2048), 1))
    pos = (lax.broadcasted_iota(jnp.int32, (_QROWS, _QCOLS), 0) * _QCOLS
           + lax.broadcasted_iota(jnp.int32, (_QROWS, _QCOLS), 1))
    evenv = (lax.broadcasted_iota(jnp.int32, (8, 128), 0) % 2) == 0
    pcol = lax.broadcasted_iota(jnp.int32, (8, 1), 0) // 2
    zf = jnp.float32(0.0)

    def col4(vals):
        return jnp.where(pcol == 0, vals[0],
                         jnp.where(pcol == 1, vals[1],
                                   jnp.where(pcol == 2, vals[2], vals[3])))

    def centroid(b, last_b):
        return tuple(xs_smem[(b * 3 + c) * _NVOX + last_b] for c in range(3))

    dists0 = jnp.full((8, 2048), 1e10, jnp.float32)
    lasts0 = tuple(jnp.int32(0) for _ in range(_B))
    nx0 = tuple(tuple(jnp.zeros((_QROWS, _QCOLS), jnp.float32) for _ in range(3))
                for _ in range(_B))
    ix0 = tuple(jnp.zeros((_QROWS, _QCOLS), jnp.int32) for _ in range(_B))

    def body(i, carry):
        dists, lasts, nxa, ixa = carry
        oh_prev = pos == (i - 1)
        oh_cur = pos == i
        cs = [centroid(b, lasts[b]) for b in range(_B)]
        cxc = col4([cs[b][0] for b in range(_B)])
        cyc = col4([cs[b][1] for b in range(_B)])
        czc = col4([cs[b][2] for b in range(_B)])
        dx = X - cxc
        dy = Y - cyc
        dz = Z - czc
        d = dx * dx + dy * dy + dz * dz
        db = jnp.minimum(dists, d)
        db3 = db.reshape(_B, 2, 2048)
        mx4 = jnp.max(db3, axis=(1, 2), keepdims=True)
        mxb = jnp.broadcast_to(mx4, (_B, 2, 2048)).reshape(8, 2048)
        sel = jnp.where(db == mxb, jdx, _NVOX)
        far4 = jnp.min(sel.reshape(_B, 2, 2048), axis=(1, 2))
        fars = tuple(far4[b] for b in range(_B))
        nnx = tuple(tuple(nxa[b][c] + jnp.where(oh_prev, cs[b][c], zf)
                          for c in range(3)) for b in range(_B))
        nix = tuple(ixa[b] + jnp.where(oh_cur, fars[b], jnp.int32(0))
                    for b in range(_B))
        return db, fars, nnx, nix

    dists, lasts, nxa, ixa = lax.fori_loop(
        1, _NPOINTS, body, (dists0, lasts0, nx0, ix0))

    oh_last = pos == (_NPOINTS - 1)
    for b in range(_B):
        cx, cy, cz = centroid(b, lasts[b])
        for c, v in enumerate((cx, cy, cz)):
            nxyz_ref[b, c] = nxa[b][c] + jnp.where(oh_last, v, zf)
        idx_ref[b] = ixa[b]


_fps_call = pl.pallas_call(
    _fps_body,
    in_specs=[
        pl.BlockSpec(memory_space=pltpu.VMEM),
        pl.BlockSpec(memory_space=pltpu.SMEM),
    ],
    out_shape=(
        jax.ShapeDtypeStruct((_B, 3, _QROWS, _QCOLS), jnp.float32),
        jax.ShapeDtypeStruct((_B, _QROWS, _QCOLS), jnp.int32),
    ),
)


@functools.cache
def _make_sc_kernel():
    return pl.kernel(
        _sc_body,
        out_type=(
            jax.ShapeDtypeStruct((_B * _NPOINTS * 3,), jnp.float32),
            jax.ShapeDtypeStruct((_B * _NPOINTS * 3,), jnp.float32),
            jax.ShapeDtypeStruct((_B * 3 * _NPOINTS * _NSAMPLE,), jnp.float32),
            jax.ShapeDtypeStruct((_B * 6 * _NPOINTS * _NSAMPLE,), jnp.float32),
        ),
        mesh=plsc.VectorSubcoreMesh(core_axis_name="c", subcore_axis_name="s",
                                    num_cores=2, num_subcores=16),
        compiler_params=pltpu.CompilerParams(needs_layout_passes=False),
        scratch_types=[
            pltpu.VMEM((_NALL,), jnp.float32),
            pltpu.VMEM((_NALL,), jnp.float32),
            pltpu.VMEM((_NALL,), jnp.float32),
            pltpu.VMEM((_NALL,), jnp.float32),
            pltpu.VMEM((_NALL,), jnp.float32),
            pltpu.VMEM((_NALL,), jnp.float32),
            pltpu.VMEM((_NVOX,), jnp.float32),
            pltpu.VMEM((_NVOX,), jnp.float32),
            pltpu.VMEM((_NVOX,), jnp.float32),
            pltpu.VMEM((16,), jnp.int32),
            pltpu.VMEM((48,), jnp.float32),
            pltpu.VMEM((64,), jnp.int32),
            pltpu.VMEM((3 * 16 * _NSAMPLE,), jnp.float32),
            pltpu.VMEM((3 * 16 * _NSAMPLE,), jnp.float32),
            pltpu.VMEM((48,), jnp.float32),
            pltpu.VMEM((48,), jnp.float32),
            pltpu.SMEM((1,), jnp.int32),
        ],
    )


def _sc_body(xa, nrm, nvox, nxc, fpsi,
             nxyz_o, nnorm_o, feat_o, nfn_o,
             Xr, Yr, Zr, NXr, NYr, NZr, VNX, VNY, VNZ,
             fidx, nxq, idxbuf, fbuf, nbuf, nxbuf, nnbuf, cnt_ref):
    wid = lax.axis_index("s") * 2 + lax.axis_index("c")
    b = wid // 8
    r = wid % 8

    for c, ref in enumerate((Xr, Yr, Zr)):
        off = pl.multiple_of((b * 3 + c) * _NALL, _NALL)
        pltpu.sync_copy(xa.at[pl.ds(off, _NALL)], ref)
    for c, ref in enumerate((NXr, NYr, NZr)):
        off = pl.multiple_of((b * 3 + c) * _NALL, _NALL)
        pltpu.sync_copy(nrm.at[pl.ds(off, _NALL)], ref)
    for c, ref in enumerate((VNX, VNY, VNZ)):
        off = pl.multiple_of((b * 3 + c) * _NVOX, _NVOX)
        pltpu.sync_copy(nvox.at[pl.ds(off, _NVOX)], ref)
    i16 = lax.iota(jnp.int32, 16)
    zeros16 = jnp.zeros((16,), jnp.int32)

    def blk_body(blk, _):
        qg = (r + 8 * blk) * 16
        pltpu.sync_copy(fpsi.at[pl.ds(b * _NPOINTS + qg, 16)], fidx)
        for c in range(3):
            off = (b * 3 + c) * _NPOINTS + qg
            pltpu.sync_copy(nxc.at[pl.ds(off, 16)], nxq.at[pl.ds(c * 16, 16)])
        qxv = nxq[pl.ds(0, 16)]
        qyv = nxq[pl.ds(16, 16)]
        qzv = nxq[pl.ds(32, 16)]
        for qi in range(16):
            qx = qxv[qi]
            qy = qyv[qi]
            qz = qzv[qi]
            idxbuf[pl.ds(0, 16)] = zeros16

            cnt_ref[0] = 0

            def seg_body(sg, _):
                @pl.when(cnt_ref[0] < _NSAMPLE)
                def _():
                    cv0 = jnp.full((16,), cnt_ref[0], jnp.int32)

                    def chunk_body(ck, cnt_vec):
                        for u in range(2):
                            cbase = sg * 256 + ck * 32 + u * 16
                            xsv = Xr[pl.ds(cbase, 16)]
                            ysv = Yr[pl.ds(cbase, 16)]
                            zsv = Zr[pl.ds(cbase, 16)]
                            dx = xsv - qx
                            dy = ysv - qy
                            dz = zsv - qz
                            d2 = dx * dx + dy * dy + dz * dz
                            m = d2 < _R2
                            jv = i16 + cbase
                            csum = plsc.cumsum(m.astype(jnp.int32))
                            pos = jnp.minimum(cnt_vec + csum - 1, 63)
                            pos = jnp.where(m, pos, 63)
                            plsc.store_scatter(idxbuf, [pos], jv)
                            cnt_vec = cnt_vec + csum[15]
                        return cnt_vec

                    cv = lax.fori_loop(0, 8, chunk_body, cv0)
                    cnt_ref[0] = cv[15]
                return 0

            lax.fori_loop(0, _NALL // 256, seg_body, 0)
            cnt = cnt_ref[0]
            v0 = idxbuf[pl.ds(0, 16)][0]
            for k in (0, 16):
                lane = i16 + k
                vec = idxbuf[pl.ds(k, 16)]
                sel = jnp.where(lane >= cnt, v0, vec)
                gx = plsc.load_gather(Xr, [sel]) - qx
                gy = plsc.load_gather(Yr, [sel]) - qy
                gz = plsc.load_gather(Zr, [sel]) - qz
                fbuf[pl.ds(qi * _NSAMPLE + k, 16)] = gx
                fbuf[pl.ds(512 + qi * _NSAMPLE + k, 16)] = gy
                fbuf[pl.ds(1024 + qi * _NSAMPLE + k, 16)] = gz
                nbuf[pl.ds(qi * _NSAMPLE + k, 16)] = plsc.load_gather(NXr, [sel])
                nbuf[pl.ds(512 + qi * _NSAMPLE + k, 16)] = plsc.load_gather(NYr, [sel])
                nbuf[pl.ds(1024 + qi * _NSAMPLE + k, 16)] = plsc.load_gather(NZr, [sel])

        fi = fidx[pl.ds(0, 16)]
        lanes3 = i16 * 3
        for c, (tab, qv) in enumerate(((VNX, qxv), (VNY, qyv), (VNZ, qzv))):
            g = plsc.load_gather(tab, [fi])
            plsc.store_scatter(nnbuf, [lanes3 + c], g)
            plsc.store_scatter(nxbuf, [lanes3 + c], qv)
        row_off = pl.multiple_of((b * _NPOINTS + qg) * 3, 48)
        pltpu.sync_copy(nnbuf, nnorm_o.at[pl.ds(row_off, 48)])
        pltpu.sync_copy(nxbuf, nxyz_o.at[pl.ds(row_off, 48)])
        for c in range(3):
            fsrc = fbuf.at[pl.ds(c * 512, 512)]
            nsrc = nbuf.at[pl.ds(c * 512, 512)]
            foff = pl.multiple_of(((b * 3 + c) * _NPOINTS + qg) * _NSAMPLE, 512)
            pltpu.sync_copy(fsrc, feat_o.at[pl.ds(foff, 512)])
            noff = pl.multiple_of(((b * 6 + c) * _NPOINTS + qg) * _NSAMPLE, 512)
            pltpu.sync_copy(fsrc, nfn_o.at[pl.ds(noff, 512)])
            noff2 = pl.multiple_of(((b * 6 + c + 3) * _NPOINTS + qg) * _NSAMPLE, 512)
            pltpu.sync_copy(nsrc, nfn_o.at[pl.ds(noff2, 512)])
        return 0

    lax.fori_loop(0, 4, blk_body, 0)


def kernel(xyz_all, normals, xyz_voxel, normals_voxel):
    xvp = xyz_voxel.transpose(2, 0, 1).reshape(3, 8, 2048)
    xv_sm = xyz_voxel.transpose(0, 2, 1).reshape(-1)
    nx_cm, fps_i = _fps_call(xvp, xv_sm)
    nxc = nx_cm.reshape(-1)
    xa_cm = xyz_all.transpose(0, 2, 1).reshape(-1)
    nrm_f = normals.reshape(-1)
    nvox_cm = normals_voxel.transpose(0, 2, 1).reshape(-1)
    new_xyz, new_normals, feat, nfn = _make_sc_kernel()(
        xa_cm, nrm_f, nvox_cm, nxc, fps_i.reshape(-1))
    return (new_xyz.reshape(_B, _NPOINTS, 3),
            new_normals.reshape(_B, _NPOINTS, 3),
            feat.reshape(_B, 3, _NPOINTS, _NSAMPLE),
            nfn.reshape(_B, 6, _NPOINTS, _NSAMPLE))


# speedup vs baseline: 1.7284x; 1.0027x over previous
import functools

import jax
import jax.numpy as jnp
import numpy as np
from jax import lax
from jax.experimental import pallas as pl
from jax.experimental.pallas import tpu as pltpu
from jax.experimental.pallas import tpu_sc as plsc

_NPOINTS = 512
_RADIUS = 0.2
_NSAMPLE = 32
_B = 4
_NVOX = 4096
_NALL = 16384
_R2 = np.float32(_RADIUS * _RADIUS)

_ROWS = 8
_COLS = _NVOX // _ROWS
_QROWS = 8
_QCOLS = _NPOINTS // _QROWS


def _fps_body(xvp_ref, xs_smem, nxyz_ref, idx_ref):
    X = xvp_ref[0]
    Y = xvp_ref[1]
    Z = xvp_ref[2]
    jdx = ((lax.broadcasted_iota(jnp.int32, (8, 2048), 0) % 2) * 2048
           + lax.broadcasted_iota(jnp.int32, (8, 2048), 1))
    pos = (lax.broadcasted_iota(jnp.int32, (_QROWS, _QCOLS), 0) * _QCOLS
           + lax.broadcasted_iota(jnp.int32, (_QROWS, _QCOLS), 1))
    evenv = (lax.broadcasted_iota(jnp.int32, (8, 128), 0) % 2) == 0
    pcol = lax.broadcasted_iota(jnp.int32, (8, 1), 0) // 2
    zf = jnp.float32(0.0)

    def col4(vals):
        return jnp.where(pcol == 0, vals[0],
                         jnp.where(pcol == 1, vals[1],
                                   jnp.where(pcol == 2, vals[2], vals[3])))

    def centroid(b, last_b):
        return tuple(xs_smem[(b * 3 + c) * _NVOX + last_b] for c in range(3))

    dists0 = jnp.full((8, 2048), 1e10, jnp.float32)
    lasts0 = tuple(jnp.int32(0) for _ in range(_B))
    nx0 = tuple(tuple(jnp.zeros((_QROWS, _QCOLS), jnp.float32) for _ in range(3))
                for _ in range(_B))
    ix0 = tuple(jnp.zeros((_QROWS, _QCOLS), jnp.int32) for _ in range(_B))

    def body(i, carry):
        dists, lasts, nxa, ixa = carry
        oh_prev = pos == (i - 1)
        oh_cur = pos == i
        cs = [centroid(b, lasts[b]) for b in range(_B)]
        cxc = col4([cs[b][0] for b in range(_B)])
        cyc = col4([cs[b][1] for b in range(_B)])
        czc = col4([cs[b][2] for b in range(_B)])
        dx = X - cxc
        dy = Y - cyc
        dz = Z - czc
        d = dx * dx + dy * dy + dz * dz
        db = jnp.minimum(dists, d)
        db3 = db.reshape(_B, 2, 2048)
        mx4 = jnp.max(db3, axis=(1, 2), keepdims=True)
        mxb = jnp.broadcast_to(mx4, (_B, 2, 2048)).reshape(8, 2048)
        sel = jnp.where(db == mxb, jdx, _NVOX)
        far4 = jnp.min(sel.reshape(_B, 2, 2048), axis=(1, 2))
        fars = tuple(far4[b] for b in range(_B))
        nnx = tuple(tuple(nxa[b][c] + jnp.where(oh_prev, cs[b][c], zf)
                          for c in range(3)) for b in range(_B))
        nix = tuple(ixa[b] + jnp.where(oh_cur, fars[b], jnp.int32(0))
                    for b in range(_B))
        return db, fars, nnx, nix

    dists, lasts, nxa, ixa = lax.fori_loop(
        1, _NPOINTS, body, (dists0, lasts0, nx0, ix0))

    oh_last = pos == (_NPOINTS - 1)
    for b in range(_B):
        cx, cy, cz = centroid(b, lasts[b])
        for c, v in enumerate((cx, cy, cz)):
            nxyz_ref[b, c] = nxa[b][c] + jnp.where(oh_last, v, zf)
        idx_ref[b] = ixa[b]


_fps_call = pl.pallas_call(
    _fps_body,
    in_specs=[
        pl.BlockSpec(memory_space=pltpu.VMEM),
        pl.BlockSpec(memory_space=pltpu.SMEM),
    ],
    out_shape=(
        jax.ShapeDtypeStruct((_B, 3, _QROWS, _QCOLS), jnp.float32),
        jax.ShapeDtypeStruct((_B, _QROWS, _QCOLS), jnp.int32),
    ),
)


@functools.cache
def _make_sc_kernel():
    return pl.kernel(
        _sc_body,
        out_type=(
            jax.ShapeDtypeStruct((_B * _NPOINTS * 3,), jnp.float32),
            jax.ShapeDtypeStruct((_B * _NPOINTS * 3,), jnp.float32),
            jax.ShapeDtypeStruct((_B * 3 * _NPOINTS * _NSAMPLE,), jnp.float32),
            jax.ShapeDtypeStruct((_B * 6 * _NPOINTS * _NSAMPLE,), jnp.float32),
        ),
        mesh=plsc.VectorSubcoreMesh(core_axis_name="c", subcore_axis_name="s",
                                    num_cores=2, num_subcores=16),
        compiler_params=pltpu.CompilerParams(needs_layout_passes=False),
        scratch_types=[
            pltpu.VMEM((_NALL,), jnp.float32),
            pltpu.VMEM((_NALL,), jnp.float32),
            pltpu.VMEM((_NALL,), jnp.float32),
            pltpu.VMEM((_NALL,), jnp.float32),
            pltpu.VMEM((_NALL,), jnp.float32),
            pltpu.VMEM((_NALL,), jnp.float32),
            pltpu.VMEM((_NVOX,), jnp.float32),
            pltpu.VMEM((_NVOX,), jnp.float32),
            pltpu.VMEM((_NVOX,), jnp.float32),
            pltpu.VMEM((16,), jnp.int32),
            pltpu.VMEM((48,), jnp.float32),
            pltpu.VMEM((64,), jnp.int32),
            pltpu.VMEM((3 * 16 * _NSAMPLE,), jnp.float32),
            pltpu.VMEM((3 * 16 * _NSAMPLE,), jnp.float32),
            pltpu.VMEM((48,), jnp.float32),
            pltpu.VMEM((48,), jnp.float32),
            pltpu.SMEM((1,), jnp.int32),
        ],
    )


def _sc_body(xa, nrm, nvox, nxc, fpsi,
             nxyz_o, nnorm_o, feat_o, nfn_o,
             Xr, Yr, Zr, NXr, NYr, NZr, VNX, VNY, VNZ,
             fidx, nxq, idxbuf, fbuf, nbuf, nxbuf, nnbuf, cnt_ref):
    wid = lax.axis_index("s") * 2 + lax.axis_index("c")
    b = wid // 8
    r = wid % 8

    for c, ref in enumerate((Xr, Yr, Zr)):
        off = pl.multiple_of((b * 3 + c) * _NALL, _NALL)
        pltpu.sync_copy(xa.at[pl.ds(off, _NALL)], ref)
    for c, ref in enumerate((NXr, NYr, NZr)):
        off = pl.multiple_of((b * 3 + c) * _NALL, _NALL)
        pltpu.sync_copy(nrm.at[pl.ds(off, _NALL)], ref)
    for c, ref in enumerate((VNX, VNY, VNZ)):
        off = pl.multiple_of((b * 3 + c) * _NVOX, _NVOX)
        pltpu.sync_copy(nvox.at[pl.ds(off, _NVOX)], ref)
    i16 = lax.iota(jnp.int32, 16)
    zeros16 = jnp.zeros((16,), jnp.int32)

    def blk_body(blk, _):
        qg = (r + 8 * blk) * 16
        pltpu.sync_copy(fpsi.at[pl.ds(b * _NPOINTS + qg, 16)], fidx)
        for c in range(3):
            off = (b * 3 + c) * _NPOINTS + qg
            pltpu.sync_copy(nxc.at[pl.ds(off, 16)], nxq.at[pl.ds(c * 16, 16)])
        qxv = nxq[pl.ds(0, 16)]
        qyv = nxq[pl.ds(16, 16)]
        qzv = nxq[pl.ds(32, 16)]
        for qi in range(16):
            qx = qxv[qi]
            qy = qyv[qi]
            qz = qzv[qi]
            idxbuf[pl.ds(0, 16)] = zeros16

            cnt_ref[0] = 0

            def seg_body(sg, _):
                @pl.when(cnt_ref[0] < _NSAMPLE)
                def _():
                    cv0 = jnp.full((16,), cnt_ref[0], jnp.int32)

                    def chunk_body(ck, cnt_vec):
                        for u in range(2):
                            cbase = sg * 256 + ck * 32 + u * 16
                            xsv = Xr[pl.ds(cbase, 16)]
                            ysv = Yr[pl.ds(cbase, 16)]
                            zsv = Zr[pl.ds(cbase, 16)]
                            dx = xsv - qx
                            dy = ysv - qy
                            dz = zsv - qz
                            d2 = dx * dx + dy * dy + dz * dz
                            m = d2 < _R2
                            jv = i16 + cbase
                            csum = plsc.cumsum(m.astype(jnp.int32))
                            pos = jnp.minimum(cnt_vec + csum - 1, 63)
                            pos = jnp.where(m, pos, 63)
                            plsc.store_scatter(idxbuf, [pos], jv)
                            cnt_vec = cnt_vec + plsc.all_reduce_population_count(m)
                        return cnt_vec

                    cv = lax.fori_loop(0, 8, chunk_body, cv0)
                    cnt_ref[0] = cv[15]
                return 0

            lax.fori_loop(0, _NALL // 256, seg_body, 0)
            cnt = cnt_ref[0]
            v0 = idxbuf[pl.ds(0, 16)][0]
            for k in (0, 16):
                lane = i16 + k
                vec = idxbuf[pl.ds(k, 16)]
                sel = jnp.where(lane >= cnt, v0, vec)
                gx = plsc.load_gather(Xr, [sel]) - qx
                gy = plsc.load_gather(Yr, [sel]) - qy
                gz = plsc.load_gather(Zr, [sel]) - qz
                fbuf[pl.ds(qi * _NSAMPLE + k, 16)] = gx
                fbuf[pl.ds(512 + qi * _NSAMPLE + k, 16)] = gy
                fbuf[pl.ds(1024 + qi * _NSAMPLE + k, 16)] = gz
                nbuf[pl.ds(qi * _NSAMPLE + k, 16)] = plsc.load_gather(NXr, [sel])
                nbuf[pl.ds(512 + qi * _NSAMPLE + k, 16)] = plsc.load_gather(NYr, [sel])
                nbuf[pl.ds(1024 + qi * _NSAMPLE + k, 16)] = plsc.load_gather(NZr, [sel])

        fi = fidx[pl.ds(0, 16)]
        lanes3 = i16 * 3
        for c, (tab, qv) in enumerate(((VNX, qxv), (VNY, qyv), (VNZ, qzv))):
            g = plsc.load_gather(tab, [fi])
            plsc.store_scatter(nnbuf, [lanes3 + c], g)
            plsc.store_scatter(nxbuf, [lanes3 + c], qv)
        row_off = pl.multiple_of((b * _NPOINTS + qg) * 3, 48)
        pltpu.sync_copy(nnbuf, nnorm_o.at[pl.ds(row_off, 48)])
        pltpu.sync_copy(nxbuf, nxyz_o.at[pl.ds(row_off, 48)])
        for c in range(3):
            fsrc = fbuf.at[pl.ds(c * 512, 512)]
            nsrc = nbuf.at[pl.ds(c * 512, 512)]
            foff = pl.multiple_of(((b * 3 + c) * _NPOINTS + qg) * _NSAMPLE, 512)
            pltpu.sync_copy(fsrc, feat_o.at[pl.ds(foff, 512)])
            noff = pl.multiple_of(((b * 6 + c) * _NPOINTS + qg) * _NSAMPLE, 512)
            pltpu.sync_copy(fsrc, nfn_o.at[pl.ds(noff, 512)])
            noff2 = pl.multiple_of(((b * 6 + c + 3) * _NPOINTS + qg) * _NSAMPLE, 512)
            pltpu.sync_copy(nsrc, nfn_o.at[pl.ds(noff2, 512)])
        return 0

    lax.fori_loop(0, 4, blk_body, 0)


def kernel(xyz_all, normals, xyz_voxel, normals_voxel):
    xvp = xyz_voxel.transpose(2, 0, 1).reshape(3, 8, 2048)
    xv_sm = xyz_voxel.transpose(0, 2, 1).reshape(-1)
    nx_cm, fps_i = _fps_call(xvp, xv_sm)
    nxc = nx_cm.reshape(-1)
    xa_cm = xyz_all.transpose(0, 2, 1).reshape(-1)
    nrm_f = normals.reshape(-1)
    nvox_cm = normals_voxel.transpose(0, 2, 1).reshape(-1)
    new_xyz, new_normals, feat, nfn = _make_sc_kernel()(
        xa_cm, nrm_f, nvox_cm, nxc, fps_i.reshape(-1))
    return (new_xyz.reshape(_B, _NPOINTS, 3),
            new_normals.reshape(_B, _NPOINTS, 3),
            feat.reshape(_B, 3, _NPOINTS, _NSAMPLE),
            nfn.reshape(_B, 6, _NPOINTS, _NSAMPLE))


# speedup vs baseline: 1.7718x; 1.0251x over previous
import functools

import jax
import jax.numpy as jnp
import numpy as np
from jax import lax
from jax.experimental import pallas as pl
from jax.experimental.pallas import tpu as pltpu
from jax.experimental.pallas import tpu_sc as plsc

_NPOINTS = 512
_RADIUS = 0.2
_NSAMPLE = 32
_B = 4
_NVOX = 4096
_NALL = 16384
_R2 = np.float32(_RADIUS * _RADIUS)

_ROWS = 8
_COLS = _NVOX // _ROWS
_QROWS = 8
_QCOLS = _NPOINTS // _QROWS


def _fps_body(xvp_ref, xs_smem, nxyz_ref, idx_ref):
    X = xvp_ref[0]
    Y = xvp_ref[1]
    Z = xvp_ref[2]
    jdx = ((lax.broadcasted_iota(jnp.int32, (8, 2048), 0) % 2) * 2048
           + lax.broadcasted_iota(jnp.int32, (8, 2048), 1))
    pos = (lax.broadcasted_iota(jnp.int32, (_QROWS, _QCOLS), 0) * _QCOLS
           + lax.broadcasted_iota(jnp.int32, (_QROWS, _QCOLS), 1))
    evenv = (lax.broadcasted_iota(jnp.int32, (8, 128), 0) % 2) == 0
    pcol = lax.broadcasted_iota(jnp.int32, (8, 1), 0) // 2
    zf = jnp.float32(0.0)

    def col4(vals):
        return jnp.where(pcol == 0, vals[0],
                         jnp.where(pcol == 1, vals[1],
                                   jnp.where(pcol == 2, vals[2], vals[3])))

    def centroid(b, last_b):
        return tuple(xs_smem[(b * 3 + c) * _NVOX + last_b] for c in range(3))

    dists0 = jnp.full((8, 2048), 1e10, jnp.float32)
    lasts0 = tuple(jnp.int32(0) for _ in range(_B))
    nx0 = tuple(tuple(jnp.zeros((_QROWS, _QCOLS), jnp.float32) for _ in range(3))
                for _ in range(_B))
    ix0 = tuple(jnp.zeros((_QROWS, _QCOLS), jnp.int32) for _ in range(_B))

    def body(i, carry):
        dists, lasts, nxa, ixa = carry
        oh_prev = pos == (i - 1)
        oh_cur = pos == i
        cs = [centroid(b, lasts[b]) for b in range(_B)]
        cxc = col4([cs[b][0] for b in range(_B)])
        cyc = col4([cs[b][1] for b in range(_B)])
        czc = col4([cs[b][2] for b in range(_B)])
        dx = X - cxc
        dy = Y - cyc
        dz = Z - czc
        d = dx * dx + dy * dy + dz * dz
        db = jnp.minimum(dists, d)
        db3 = db.reshape(_B, 2, 2048)
        mx4 = jnp.max(db3, axis=(1, 2), keepdims=True)
        mxb = jnp.broadcast_to(mx4, (_B, 2, 2048)).reshape(8, 2048)
        sel = jnp.where(db == mxb, jdx, _NVOX)
        far4 = jnp.min(sel.reshape(_B, 2, 2048), axis=(1, 2))
        fars = tuple(far4[b] for b in range(_B))
        nnx = tuple(tuple(nxa[b][c] + jnp.where(oh_prev, cs[b][c], zf)
                          for c in range(3)) for b in range(_B))
        nix = tuple(ixa[b] + jnp.where(oh_cur, fars[b], jnp.int32(0))
                    for b in range(_B))
        return db, fars, nnx, nix

    dists, lasts, nxa, ixa = lax.fori_loop(
        1, _NPOINTS, body, (dists0, lasts0, nx0, ix0))

    oh_last = pos == (_NPOINTS - 1)
    for b in range(_B):
        cx, cy, cz = centroid(b, lasts[b])
        for c, v in enumerate((cx, cy, cz)):
            nxyz_ref[b, c] = nxa[b][c] + jnp.where(oh_last, v, zf)
        idx_ref[b] = ixa[b]


_fps_call = pl.pallas_call(
    _fps_body,
    in_specs=[
        pl.BlockSpec(memory_space=pltpu.VMEM),
        pl.BlockSpec(memory_space=pltpu.SMEM),
    ],
    out_shape=(
        jax.ShapeDtypeStruct((_B, 3, _QROWS, _QCOLS), jnp.float32),
        jax.ShapeDtypeStruct((_B, _QROWS, _QCOLS), jnp.int32),
    ),
)


@functools.cache
def _make_sc_kernel():
    return pl.kernel(
        _sc_body,
        out_type=(
            jax.ShapeDtypeStruct((_B * _NPOINTS * 3,), jnp.float32),
            jax.ShapeDtypeStruct((_B * _NPOINTS * 3,), jnp.float32),
            jax.ShapeDtypeStruct((_B * 3 * _NPOINTS * _NSAMPLE,), jnp.float32),
            jax.ShapeDtypeStruct((_B * 6 * _NPOINTS * _NSAMPLE,), jnp.float32),
        ),
        mesh=plsc.VectorSubcoreMesh(core_axis_name="c", subcore_axis_name="s",
                                    num_cores=2, num_subcores=16),
        compiler_params=pltpu.CompilerParams(needs_layout_passes=False),
        scratch_types=[
            pltpu.VMEM((_NALL,), jnp.float32),
            pltpu.VMEM((_NALL,), jnp.float32),
            pltpu.VMEM((_NALL,), jnp.float32),
            pltpu.VMEM((_NALL,), jnp.float32),
            pltpu.VMEM((_NALL,), jnp.float32),
            pltpu.VMEM((_NALL,), jnp.float32),
            pltpu.VMEM((_NVOX,), jnp.float32),
            pltpu.VMEM((_NVOX,), jnp.float32),
            pltpu.VMEM((_NVOX,), jnp.float32),
            pltpu.VMEM((16,), jnp.int32),
            pltpu.VMEM((48,), jnp.float32),
            pltpu.VMEM((512,), jnp.int32),
            pltpu.VMEM((3 * 16 * _NSAMPLE,), jnp.float32),
            pltpu.VMEM((3 * 16 * _NSAMPLE,), jnp.float32),
            pltpu.VMEM((48,), jnp.float32),
            pltpu.VMEM((48,), jnp.float32),
            pltpu.SMEM((1,), jnp.int32),
        ],
    )


def _sc_body(xa, nrm, nvox, nxc, fpsi,
             nxyz_o, nnorm_o, feat_o, nfn_o,
             Xr, Yr, Zr, NXr, NYr, NZr, VNX, VNY, VNZ,
             fidx, nxq, idxbuf, fbuf, nbuf, nxbuf, nnbuf, cnt_ref):
    wid = lax.axis_index("s") * 2 + lax.axis_index("c")
    b = wid // 8
    r = wid % 8

    for c, ref in enumerate((Xr, Yr, Zr)):
        off = pl.multiple_of((b * 3 + c) * _NALL, _NALL)
        pltpu.sync_copy(xa.at[pl.ds(off, _NALL)], ref)
    for c, ref in enumerate((NXr, NYr, NZr)):
        off = pl.multiple_of((b * 3 + c) * _NALL, _NALL)
        pltpu.sync_copy(nrm.at[pl.ds(off, _NALL)], ref)
    for c, ref in enumerate((VNX, VNY, VNZ)):
        off = pl.multiple_of((b * 3 + c) * _NVOX, _NVOX)
        pltpu.sync_copy(nvox.at[pl.ds(off, _NVOX)], ref)
    i16 = lax.iota(jnp.int32, 16)
    zeros16 = jnp.zeros((16,), jnp.int32)

    def blk_body(blk, _):
        qg = (r + 8 * blk) * 16
        pltpu.sync_copy(fpsi.at[pl.ds(b * _NPOINTS + qg, 16)], fidx)
        for c in range(3):
            off = (b * 3 + c) * _NPOINTS + qg
            pltpu.sync_copy(nxc.at[pl.ds(off, 16)], nxq.at[pl.ds(c * 16, 16)])
        qxv = nxq[pl.ds(0, 16)]
        qyv = nxq[pl.ds(16, 16)]
        qzv = nxq[pl.ds(32, 16)]
        for qi in range(16):
            qx = qxv[qi]
            qy = qyv[qi]
            qz = qzv[qi]
            idxbuf[pl.ds(0, 16)] = zeros16

            cnt_ref[0] = 0

            def seg_body(sg, _):
                @pl.when(cnt_ref[0] < _NSAMPLE)
                def _():
                    cv0 = jnp.full((16,), cnt_ref[0], jnp.int32)

                    def chunk_body(ck, cnt_vec):
                        for u in range(4):
                            cbase = sg * 256 + ck * 64 + u * 16
                            xsv = Xr[pl.ds(cbase, 16)]
                            ysv = Yr[pl.ds(cbase, 16)]
                            zsv = Zr[pl.ds(cbase, 16)]
                            dx = xsv - qx
                            dy = ysv - qy
                            dz = zsv - qz
                            d2 = dx * dx + dy * dy + dz * dz
                            m = d2 < _R2
                            jv = i16 + cbase
                            csum = plsc.cumsum(m.astype(jnp.int32))
                            pos = jnp.where(m, cnt_vec + csum - 1, 511)
                            plsc.store_scatter(idxbuf, [pos], jv)
                            cnt_vec = cnt_vec + plsc.all_reduce_population_count(m)
                        return cnt_vec

                    cv = lax.fori_loop(0, 4, chunk_body, cv0)
                    cnt_ref[0] = cv[15]
                return 0

            lax.fori_loop(0, _NALL // 256, seg_body, 0)
            cnt = cnt_ref[0]
            v0 = idxbuf[pl.ds(0, 16)][0]
            for k in (0, 16):
                lane = i16 + k
                vec = idxbuf[pl.ds(k, 16)]
                sel = jnp.where(lane >= cnt, v0, vec)
                gx = plsc.load_gather(Xr, [sel]) - qx
                gy = plsc.load_gather(Yr, [sel]) - qy
                gz = plsc.load_gather(Zr, [sel]) - qz
                fbuf[pl.ds(qi * _NSAMPLE + k, 16)] = gx
                fbuf[pl.ds(512 + qi * _NSAMPLE + k, 16)] = gy
                fbuf[pl.ds(1024 + qi * _NSAMPLE + k, 16)] = gz
                nbuf[pl.ds(qi * _NSAMPLE + k, 16)] = plsc.load_gather(NXr, [sel])
                nbuf[pl.ds(512 + qi * _NSAMPLE + k, 16)] = plsc.load_gather(NYr, [sel])
                nbuf[pl.ds(1024 + qi * _NSAMPLE + k, 16)] = plsc.load_gather(NZr, [sel])

        fi = fidx[pl.ds(0, 16)]
        lanes3 = i16 * 3
        for c, (tab, qv) in enumerate(((VNX, qxv), (VNY, qyv), (VNZ, qzv))):
            g = plsc.load_gather(tab, [fi])
            plsc.store_scatter(nnbuf, [lanes3 + c], g)
            plsc.store_scatter(nxbuf, [lanes3 + c], qv)
        row_off = pl.multiple_of((b * _NPOINTS + qg) * 3, 48)
        pltpu.sync_copy(nnbuf, nnorm_o.at[pl.ds(row_off, 48)])
        pltpu.sync_copy(nxbuf, nxyz_o.at[pl.ds(row_off, 48)])
        for c in range(3):
            fsrc = fbuf.at[pl.ds(c * 512, 512)]
            nsrc = nbuf.at[pl.ds(c * 512, 512)]
            foff = pl.multiple_of(((b * 3 + c) * _NPOINTS + qg) * _NSAMPLE, 512)
            pltpu.sync_copy(fsrc, feat_o.at[pl.ds(foff, 512)])
            noff = pl.multiple_of(((b * 6 + c) * _NPOINTS + qg) * _NSAMPLE, 512)
            pltpu.sync_copy(fsrc, nfn_o.at[pl.ds(noff, 512)])
            noff2 = pl.multiple_of(((b * 6 + c + 3) * _NPOINTS + qg) * _NSAMPLE, 512)
            pltpu.sync_copy(nsrc, nfn_o.at[pl.ds(noff2, 512)])
        return 0

    lax.fori_loop(0, 4, blk_body, 0)


def kernel(xyz_all, normals, xyz_voxel, normals_voxel):
    xvp = xyz_voxel.transpose(2, 0, 1).reshape(3, 8, 2048)
    xv_sm = xyz_voxel.transpose(0, 2, 1).reshape(-1)
    nx_cm, fps_i = _fps_call(xvp, xv_sm)
    nxc = nx_cm.reshape(-1)
    xa_cm = xyz_all.transpose(0, 2, 1).reshape(-1)
    nrm_f = normals.reshape(-1)
    nvox_cm = normals_voxel.transpose(0, 2, 1).reshape(-1)
    new_xyz, new_normals, feat, nfn = _make_sc_kernel()(
        xa_cm, nrm_f, nvox_cm, nxc, fps_i.reshape(-1))
    return (new_xyz.reshape(_B, _NPOINTS, 3),
            new_normals.reshape(_B, _NPOINTS, 3),
            feat.reshape(_B, 3, _NPOINTS, _NSAMPLE),
            nfn.reshape(_B, 6, _NPOINTS, _NSAMPLE))


# speedup vs baseline: 1.8535x; 1.0461x over previous
import functools

import jax
import jax.numpy as jnp
import numpy as np
from jax import lax
from jax.experimental import pallas as pl
from jax.experimental.pallas import tpu as pltpu
from jax.experimental.pallas import tpu_sc as plsc

_NPOINTS = 512
_RADIUS = 0.2
_NSAMPLE = 32
_B = 4
_NVOX = 4096
_NALL = 16384
_R2 = np.float32(_RADIUS * _RADIUS)

_ROWS = 8
_COLS = _NVOX // _ROWS
_QROWS = 8
_QCOLS = _NPOINTS // _QROWS


def _fps_body(xvp_ref, xs_smem, nxyz_ref, idx_ref):
    X = xvp_ref[0]
    Y = xvp_ref[1]
    Z = xvp_ref[2]
    jdx = ((lax.broadcasted_iota(jnp.int32, (8, 2048), 0) % 2) * 2048
           + lax.broadcasted_iota(jnp.int32, (8, 2048), 1))
    pos = (lax.broadcasted_iota(jnp.int32, (_QROWS, _QCOLS), 0) * _QCOLS
           + lax.broadcasted_iota(jnp.int32, (_QROWS, _QCOLS), 1))
    evenv = (lax.broadcasted_iota(jnp.int32, (8, 128), 0) % 2) == 0
    pcol = lax.broadcasted_iota(jnp.int32, (8, 1), 0) // 2
    zf = jnp.float32(0.0)

    def col4(vals):
        return jnp.where(pcol == 0, vals[0],
                         jnp.where(pcol == 1, vals[1],
                                   jnp.where(pcol == 2, vals[2], vals[3])))

    def centroid(b, last_b):
        return tuple(xs_smem[(b * 3 + c) * _NVOX + last_b] for c in range(3))

    dists0 = jnp.full((8, 2048), 1e10, jnp.float32)
    lasts0 = tuple(jnp.int32(0) for _ in range(_B))
    nx0 = tuple(tuple(jnp.zeros((_QROWS, _QCOLS), jnp.float32) for _ in range(3))
                for _ in range(_B))
    ix0 = tuple(jnp.zeros((_QROWS, _QCOLS), jnp.int32) for _ in range(_B))

    def body(i, carry):
        dists, lasts, nxa, ixa = carry
        oh_prev = pos == (i - 1)
        oh_cur = pos == i
        cs = [centroid(b, lasts[b]) for b in range(_B)]
        cxc = col4([cs[b][0] for b in range(_B)])
        cyc = col4([cs[b][1] for b in range(_B)])
        czc = col4([cs[b][2] for b in range(_B)])
        dx = X - cxc
        dy = Y - cyc
        dz = Z - czc
        d = dx * dx + dy * dy + dz * dz
        db = jnp.minimum(dists, d)
        db3 = db.reshape(_B, 2, 2048)
        mx4 = jnp.max(db3, axis=(1, 2), keepdims=True)
        mxb = jnp.broadcast_to(mx4, (_B, 2, 2048)).reshape(8, 2048)
        sel = jnp.where(db == mxb, jdx, _NVOX)
        far4 = jnp.min(sel.reshape(_B, 2, 2048), axis=(1, 2))
        fars = tuple(far4[b] for b in range(_B))
        nnx = tuple(tuple(nxa[b][c] + jnp.where(oh_prev, cs[b][c], zf)
                          for c in range(3)) for b in range(_B))
        nix = tuple(ixa[b] + jnp.where(oh_cur, fars[b], jnp.int32(0))
                    for b in range(_B))
        return db, fars, nnx, nix

    dists, lasts, nxa, ixa = lax.fori_loop(
        1, _NPOINTS, body, (dists0, lasts0, nx0, ix0))

    oh_last = pos == (_NPOINTS - 1)
    for b in range(_B):
        cx, cy, cz = centroid(b, lasts[b])
        for c, v in enumerate((cx, cy, cz)):
            nxyz_ref[b, c] = nxa[b][c] + jnp.where(oh_last, v, zf)
        idx_ref[b] = ixa[b]


_fps_call = pl.pallas_call(
    _fps_body,
    in_specs=[
        pl.BlockSpec(memory_space=pltpu.VMEM),
        pl.BlockSpec(memory_space=pltpu.SMEM),
    ],
    out_shape=(
        jax.ShapeDtypeStruct((_B, 3, _QROWS, _QCOLS), jnp.float32),
        jax.ShapeDtypeStruct((_B, _QROWS, _QCOLS), jnp.int32),
    ),
)


@functools.cache
def _make_sc_kernel():
    return pl.kernel(
        _sc_body,
        out_type=(
            jax.ShapeDtypeStruct((_B * _NPOINTS * 3,), jnp.float32),
            jax.ShapeDtypeStruct((_B * _NPOINTS * 3,), jnp.float32),
            jax.ShapeDtypeStruct((_B * 3 * _NPOINTS * _NSAMPLE,), jnp.float32),
            jax.ShapeDtypeStruct((_B * 6 * _NPOINTS * _NSAMPLE,), jnp.float32),
        ),
        mesh=plsc.VectorSubcoreMesh(core_axis_name="c", subcore_axis_name="s",
                                    num_cores=2, num_subcores=16),
        compiler_params=pltpu.CompilerParams(needs_layout_passes=False),
        scratch_types=[
            pltpu.VMEM((_NALL,), jnp.float32),
            pltpu.VMEM((_NALL,), jnp.float32),
            pltpu.VMEM((_NALL,), jnp.float32),
            pltpu.VMEM((_NALL,), jnp.float32),
            pltpu.VMEM((_NALL,), jnp.float32),
            pltpu.VMEM((_NALL,), jnp.float32),
            pltpu.VMEM((_NVOX,), jnp.float32),
            pltpu.VMEM((_NVOX,), jnp.float32),
            pltpu.VMEM((_NVOX,), jnp.float32),
            pltpu.VMEM((16,), jnp.int32),
            pltpu.VMEM((48,), jnp.float32),
            pltpu.VMEM((512,), jnp.int32),
            pltpu.VMEM((3 * 16 * _NSAMPLE,), jnp.float32),
            pltpu.VMEM((3 * 16 * _NSAMPLE,), jnp.float32),
            pltpu.VMEM((48,), jnp.float32),
            pltpu.VMEM((48,), jnp.float32),
            pltpu.SMEM((1,), jnp.int32),
        ],
    )


def _sc_body(xa, nrm, nvox, nxc, fpsi,
             nxyz_o, nnorm_o, feat_o, nfn_o,
             Xr, Yr, Zr, NXr, NYr, NZr, VNX, VNY, VNZ,
             fidx, nxq, idxbuf, fbuf, nbuf, nxbuf, nnbuf, cnt_ref):
    wid = lax.axis_index("s") * 2 + lax.axis_index("c")
    b = wid // 8
    r = wid % 8

    for c, ref in enumerate((Xr, Yr, Zr)):
        off = pl.multiple_of((b * 3 + c) * _NALL, _NALL)
        pltpu.sync_copy(xa.at[pl.ds(off, _NALL)], ref)
    for c, ref in enumerate((NXr, NYr, NZr)):
        off = pl.multiple_of((b * 3 + c) * _NALL, _NALL)
        pltpu.sync_copy(nrm.at[pl.ds(off, _NALL)], ref)
    for c, ref in enumerate((VNX, VNY, VNZ)):
        off = pl.multiple_of((b * 3 + c) * _NVOX, _NVOX)
        pltpu.sync_copy(nvox.at[pl.ds(off, _NVOX)], ref)
    i16 = lax.iota(jnp.int32, 16)
    zeros16 = jnp.zeros((16,), jnp.int32)

    def blk_body(blk, _):
        qg = (r + 8 * blk) * 16
        pltpu.sync_copy(fpsi.at[pl.ds(b * _NPOINTS + qg, 16)], fidx)
        for c in range(3):
            off = (b * 3 + c) * _NPOINTS + qg
            pltpu.sync_copy(nxc.at[pl.ds(off, 16)], nxq.at[pl.ds(c * 16, 16)])
        qxv = nxq[pl.ds(0, 16)]
        qyv = nxq[pl.ds(16, 16)]
        qzv = nxq[pl.ds(32, 16)]
        for qi in range(16):
            qx = qxv[qi]
            qy = qyv[qi]
            qz = qzv[qi]
            idxbuf[pl.ds(0, 16)] = zeros16

            cnt_ref[0] = 0

            def super_body(ss, _):
                @pl.when(cnt_ref[0] < _NSAMPLE)
                def _():
                    def seg_body(sg, _):
                        _scan_segment(ss * 8 + sg)
                        return 0

                    lax.fori_loop(0, 8, seg_body, 0)
                return 0

            def _scan_segment(sg):
                @pl.when(cnt_ref[0] < _NSAMPLE)
                def _():
                    cv0 = jnp.full((16,), cnt_ref[0], jnp.int32)

                    def chunk_body(ck, cnt_vec):
                        for u in range(4):
                            cbase = sg * 256 + ck * 64 + u * 16
                            xsv = Xr[pl.ds(cbase, 16)]
                            ysv = Yr[pl.ds(cbase, 16)]
                            zsv = Zr[pl.ds(cbase, 16)]
                            dx = xsv - qx
                            dy = ysv - qy
                            dz = zsv - qz
                            d2 = dx * dx + dy * dy + dz * dz
                            m = d2 < _R2
                            jv = i16 + cbase
                            csum = plsc.cumsum(m.astype(jnp.int32))
                            pos = jnp.where(m, cnt_vec + csum - 1, 511)
                            plsc.store_scatter(idxbuf, [pos], jv)
                            cnt_vec = cnt_vec + plsc.all_reduce_population_count(m)
                        return cnt_vec

                    cv = lax.fori_loop(0, 4, chunk_body, cv0)
                    cnt_ref[0] = cv[15]

            lax.fori_loop(0, 8, super_body, 0)
            cnt = cnt_ref[0]
            v0 = idxbuf[pl.ds(0, 16)][0]
            for k in (0, 16):
                lane = i16 + k
                vec = idxbuf[pl.ds(k, 16)]
                sel = jnp.where(lane >= cnt, v0, vec)
                gx = plsc.load_gather(Xr, [sel]) - qx
                gy = plsc.load_gather(Yr, [sel]) - qy
                gz = plsc.load_gather(Zr, [sel]) - qz
                fbuf[pl.ds(qi * _NSAMPLE + k, 16)] = gx
                fbuf[pl.ds(512 + qi * _NSAMPLE + k, 16)] = gy
                fbuf[pl.ds(1024 + qi * _NSAMPLE + k, 16)] = gz
                nbuf[pl.ds(qi * _NSAMPLE + k, 16)] = plsc.load_gather(NXr, [sel])
                nbuf[pl.ds(512 + qi * _NSAMPLE + k, 16)] = plsc.load_gather(NYr, [sel])
                nbuf[pl.ds(1024 + qi * _NSAMPLE + k, 16)] = plsc.load_gather(NZr, [sel])

        fi = fidx[pl.ds(0, 16)]
        lanes3 = i16 * 3
        for c, (tab, qv) in enumerate(((VNX, qxv), (VNY, qyv), (VNZ, qzv))):
            g = plsc.load_gather(tab, [fi])
            plsc.store_scatter(nnbuf, [lanes3 + c], g)
            plsc.store_scatter(nxbuf, [lanes3 + c], qv)
        row_off = pl.multiple_of((b * _NPOINTS + qg) * 3, 48)
        pltpu.sync_copy(nnbuf, nnorm_o.at[pl.ds(row_off, 48)])
        pltpu.sync_copy(nxbuf, nxyz_o.at[pl.ds(row_off, 48)])
        for c in range(3):
            fsrc = fbuf.at[pl.ds(c * 512, 512)]
            nsrc = nbuf.at[pl.ds(c * 512, 512)]
            foff = pl.multiple_of(((b * 3 + c) * _NPOINTS + qg) * _NSAMPLE, 512)
            pltpu.sync_copy(fsrc, feat_o.at[pl.ds(foff, 512)])
            noff = pl.multiple_of(((b * 6 + c) * _NPOINTS + qg) * _NSAMPLE, 512)
            pltpu.sync_copy(fsrc, nfn_o.at[pl.ds(noff, 512)])
            noff2 = pl.multiple_of(((b * 6 + c + 3) * _NPOINTS + qg) * _NSAMPLE, 512)
            pltpu.sync_copy(nsrc, nfn_o.at[pl.ds(noff2, 512)])
        return 0

    lax.fori_loop(0, 4, blk_body, 0)


def kernel(xyz_all, normals, xyz_voxel, normals_voxel):
    xvp = xyz_voxel.transpose(2, 0, 1).reshape(3, 8, 2048)
    xv_sm = xyz_voxel.transpose(0, 2, 1).reshape(-1)
    nx_cm, fps_i = _fps_call(xvp, xv_sm)
    nxc = nx_cm.reshape(-1)
    xa_cm = xyz_all.transpose(0, 2, 1).reshape(-1)
    nrm_f = normals.reshape(-1)
    nvox_cm = normals_voxel.transpose(0, 2, 1).reshape(-1)
    new_xyz, new_normals, feat, nfn = _make_sc_kernel()(
        xa_cm, nrm_f, nvox_cm, nxc, fps_i.reshape(-1))
    return (new_xyz.reshape(_B, _NPOINTS, 3),
            new_normals.reshape(_B, _NPOINTS, 3),
            feat.reshape(_B, 3, _NPOINTS, _NSAMPLE),
            nfn.reshape(_B, 6, _NPOINTS, _NSAMPLE))
